# baseline (speedup 1.0000x reference)
import jax
import jax.numpy as jnp
from jax.experimental import pallas as pl
from jax.experimental.pallas import tpu as pltpu

B = 1024
M = 200
D = 64
HOPS = 3
TBL = 128


def _hop_kernel(q_ref, gp_ref, m0_ref, m1_ref, m2_ref, m3_ref,
                soft_ref, logits_ref, u_ref):
    h = pl.program_id(1)
    w = gp_ref[...]

    @pl.when(h == 0)
    def _():
        u_ref[...] = q_ref[...]

    m_refs = (m0_ref, m1_ref, m2_ref, m3_ref)
    for hop in range(HOPS):
        @pl.when(h == hop)
        def _(hop=hop):
            u = u_ref[...]
            mh = m_refs[hop][...]
            logits = w * jnp.sum(mh * u[None, :, :], axis=1)
            mx = jnp.max(logits, axis=0, keepdims=True)
            e = jnp.exp(logits - mx)
            p = e / jnp.sum(e, axis=0, keepdims=True)
            pw = p * w
            mc = m_refs[hop + 1][...]
            o = jnp.sum(mc * pw[:, None, :], axis=0)
            u_ref[...] = u + o
            soft_ref[...] = p
            logits_ref[...] = logits


@jax.jit
def kernel(query_vector, global_pointer, m0, m1, m2, m3):
    grid = (B // TBL, HOPS)
    m0_spec = pl.BlockSpec((M, D, TBL), lambda i, h: (0, 0, i))
    m1_spec = pl.BlockSpec((M, D, TBL), lambda i, h: (0, 0, i))
    m2_spec = pl.BlockSpec(
        (M, D, TBL), lambda i, h: (0, 0, jnp.maximum(i - (h < 1), 0)))
    m3_spec = pl.BlockSpec(
        (M, D, TBL), lambda i, h: (0, 0, jnp.maximum(i - (h < 2), 0)))
    out = pl.pallas_call(
        _hop_kernel,
        grid=grid,
        in_specs=[
            pl.BlockSpec((D, TBL), lambda i, h: (0, i)),
            pl.BlockSpec((M, TBL), lambda i, h: (0, i)),
            m0_spec, m1_spec, m2_spec, m3_spec,
        ],
        out_specs=[
            pl.BlockSpec((M, TBL), lambda i, h: (0, i)),
            pl.BlockSpec((M, TBL), lambda i, h: (0, i)),
        ],
        out_shape=[
            jax.ShapeDtypeStruct((M, B), jnp.float32),
            jax.ShapeDtypeStruct((M, B), jnp.float32),
        ],
        scratch_shapes=[pltpu.VMEM((D, TBL), jnp.float32)],
    )(query_vector.T, global_pointer.T,
      jnp.transpose(m0, (1, 2, 0)), jnp.transpose(m1, (1, 2, 0)),
      jnp.transpose(m2, (1, 2, 0)), jnp.transpose(m3, (1, 2, 0)))
    return (out[0].T, out[1].T)


# speedup vs baseline: 1.1098x; 1.1098x over previous
import jax
import jax.numpy as jnp
from jax.experimental import pallas as pl

B = 1024
M = 200
D = 64
HOPS = 3
TBL = 128


def _hop_kernel(q_ref, gp_ref, m0_ref, m1_ref, m2_ref, m3_ref,
                soft_ref, logits_ref):
    u = q_ref[...]
    w = gp_ref[...]
    m_refs = (m0_ref, m1_ref, m2_ref, m3_ref)
    p = None
    logits = None
    for hop in range(HOPS):
        mh = m_refs[hop][...]
        logits = w * jnp.sum(mh * u[None, :, :], axis=1)
        mx = jnp.max(logits, axis=0, keepdims=True)
        e = jnp.exp(logits - mx)
        p = e / jnp.sum(e, axis=0, keepdims=True)
        pw = p * w
        mc = m_refs[hop + 1][...]
        o = jnp.sum(mc * pw[:, None, :], axis=0)
        u = u + o
    soft_ref[...] = p
    logits_ref[...] = logits


@jax.jit
def kernel(query_vector, global_pointer, m0, m1, m2, m3):
    grid = (B // TBL,)
    mspec = pl.BlockSpec((M, D, TBL), lambda i: (0, 0, i))
    out = pl.pallas_call(
        _hop_kernel,
        grid=grid,
        in_specs=[
            pl.BlockSpec((D, TBL), lambda i: (0, i)),
            pl.BlockSpec((M, TBL), lambda i: (0, i)),
            mspec, mspec, mspec, mspec,
        ],
        out_specs=[
            pl.BlockSpec((M, TBL), lambda i: (0, i)),
            pl.BlockSpec((M, TBL), lambda i: (0, i)),
        ],
        out_shape=[
            jax.ShapeDtypeStruct((M, B), jnp.float32),
            jax.ShapeDtypeStruct((M, B), jnp.float32),
        ],
    )(query_vector.T, global_pointer.T,
      jnp.transpose(m0, (1, 2, 0)), jnp.transpose(m1, (1, 2, 0)),
      jnp.transpose(m2, (1, 2, 0)), jnp.transpose(m3, (1, 2, 0)))
    return (out[0].T, out[1].T)
